# baseline (speedup 1.0000x reference)
import jax
import jax.numpy as jnp
from jax import lax
from jax.experimental import pallas as pl
from jax.experimental.pallas import tpu as pltpu
from jax.experimental.pallas import tpu_sc as plsc

F = 26
V = 50
D = 32
B = 16384

NC = 2
NS = 16
NW = NC * NS
BBLK = 128
BPF = B // BBLK
ITEMS = F * BPF
IPW = ITEMS // NW
TAB = D * F * V
LANES = 16


def _sc_body(inputs_t_hbm, tab_hbm, out_hbm, idx_v, tab_v, buf0, buf1,
             sem0, sem1):
    wid = lax.axis_index("s") * NC + lax.axis_index("c")
    g0 = wid * IPW

    f_lo = g0 // BPF
    f_hi = (g0 + IPW - 1) // BPF
    pltpu.async_copy(tab_hbm, tab_v, sem0)
    pltpu.async_copy(inputs_t_hbm.at[f_lo], idx_v.at[pl.ds(0, B)], sem1)
    pltpu.async_copy(inputs_t_hbm.at[f_hi], idx_v.at[pl.ds(B, B)], sem1)
    pltpu.make_async_copy(tab_hbm, tab_v, sem0).wait()
    pltpu.make_async_copy(inputs_t_hbm.at[f_lo], idx_v.at[pl.ds(0, B)],
                          sem1).wait()
    pltpu.make_async_copy(inputs_t_hbm.at[f_hi], idx_v.at[pl.ds(B, B)],
                          sem1).wait()

    def compute(g, buf):
        f = g // BPF
        b0 = (g % BPF) * BBLK
        base_off = (f - f_lo) * B + b0
        fv = f * V
        for i in range(BBLK // LANES):
            a16 = idx_v[pl.ds(base_off + LANES * i, LANES)] + fv
            a16 = (a16 & (-16)) | lax.iota(jnp.int32, LANES)
            for dd in range(0, D, 8):
                vals = [plsc.load_gather(tab_v, [a16 + (dd + k) * (F * V)])
                        for k in range(8)]
                for k in range(8):
                    buf[dd + k, pl.ds(LANES * i, LANES)] = vals[k]

    def fire(g, buf, sem):
        f = g // BPF
        b0 = (g % BPF) * BBLK
        pltpu.async_copy(buf, out_hbm.at[f, :, pl.ds(b0, BBLK)], sem)

    def drain(g, buf, sem):
        f = g // BPF
        b0 = (g % BPF) * BBLK
        pltpu.make_async_copy(buf, out_hbm.at[f, :, pl.ds(b0, BBLK)],
                              sem).wait()

    compute(g0, buf0)
    fire(g0, buf0, sem0)

    def step(t, buf, sem, nbuf, nsem):
        g = g0 + t

        @pl.when(t + 1 < IPW)
        def _():
            compute(g + 1, nbuf)
            fire(g + 1, nbuf, nsem)

        drain(g, buf, sem)

    def pair(t, carry):
        step(2 * t, buf0, sem0, buf1, sem1)
        step(2 * t + 1, buf1, sem1, buf0, sem0)
        return carry

    lax.fori_loop(0, IPW // 2, pair, 0)


@jax.jit
def _lookup(inputs_t, tab_flat):
    mesh = plsc.VectorSubcoreMesh(core_axis_name="c", subcore_axis_name="s")
    run = pl.kernel(
        _sc_body,
        out_type=jax.ShapeDtypeStruct((F, D, B), jnp.float32),
        mesh=mesh,
        scratch_types=[
            pltpu.VMEM((2 * B,), jnp.int32),
            pltpu.VMEM((TAB,), jnp.float32),
            pltpu.VMEM((D, BBLK), jnp.float32),
            pltpu.VMEM((D, BBLK), jnp.float32),
            pltpu.SemaphoreType.DMA,
            pltpu.SemaphoreType.DMA,
        ],
        compiler_params=pltpu.CompilerParams(
            use_tc_tiling_on_sc=True, needs_layout_passes=False),
    )
    return run(inputs_t, tab_flat)


def kernel(inputs, tables):
    inputs_t = inputs.T
    tab_flat = tables.transpose(2, 0, 1).reshape(TAB)
    out_t = _lookup(inputs_t, tab_flat)
    return out_t.transpose(2, 0, 1)


# speedup vs baseline: 1.0138x; 1.0138x over previous
import jax
import jax.numpy as jnp
from jax import lax
from jax.experimental import pallas as pl
from jax.experimental.pallas import tpu as pltpu
from jax.experimental.pallas import tpu_sc as plsc

F = 26
V = 50
D = 32
B = 16384

NC = 2
NS = 16
NW = NC * NS
BBLK = 128
BPF = B // BBLK
ITEMS = F * BPF
IPW = ITEMS // NW
TAB = D * F * V
LANES = 16


def _sc_body(inputs_t_hbm, tab_hbm, out_hbm, idx_v, tab_v, buf0, buf1,
             sem0, sem1):
    wid = lax.axis_index("s") * NC + lax.axis_index("c")
    g0 = wid * IPW

    f_lo = g0 // BPF
    f_hi = (g0 + IPW - 1) // BPF
    pltpu.async_copy(tab_hbm, tab_v, sem0)
    pltpu.async_copy(inputs_t_hbm.at[f_lo], idx_v.at[pl.ds(0, B)], sem1)
    pltpu.async_copy(inputs_t_hbm.at[f_hi], idx_v.at[pl.ds(B, B)], sem1)
    pltpu.make_async_copy(tab_hbm, tab_v, sem0).wait()
    pltpu.make_async_copy(inputs_t_hbm.at[f_lo], idx_v.at[pl.ds(0, B)],
                          sem1).wait()
    pltpu.make_async_copy(inputs_t_hbm.at[f_hi], idx_v.at[pl.ds(B, B)],
                          sem1).wait()

    def compute(g, buf):
        f = g // BPF
        b0 = (g % BPF) * BBLK
        base_off = (f - f_lo) * B + b0
        fv = f * V
        prev = None
        for i in range(BBLK // LANES):
            a16 = idx_v[pl.ds(base_off + LANES * i, LANES)] + fv
            for dd in range(0, D, 8):
                vals = [plsc.load_gather(tab_v, [a16 + (dd + k) * (F * V)])
                        for k in range(8)]
                if prev is not None:
                    pi, pdd, pvals = prev
                    for k in range(8):
                        buf[pdd + k, pl.ds(LANES * pi, LANES)] = pvals[k]
                prev = (i, dd, vals)
        pi, pdd, pvals = prev
        for k in range(8):
            buf[pdd + k, pl.ds(LANES * pi, LANES)] = pvals[k]

    def fire(g, buf, sem):
        f = g // BPF
        b0 = (g % BPF) * BBLK
        pltpu.async_copy(buf, out_hbm.at[f, :, pl.ds(b0, BBLK)], sem)

    def drain(g, buf, sem):
        f = g // BPF
        b0 = (g % BPF) * BBLK
        pltpu.make_async_copy(buf, out_hbm.at[f, :, pl.ds(b0, BBLK)],
                              sem).wait()

    compute(g0, buf0)
    fire(g0, buf0, sem0)

    def step(t, buf, sem, nbuf, nsem):
        g = g0 + t

        @pl.when(t + 1 < IPW)
        def _():
            compute(g + 1, nbuf)
            fire(g + 1, nbuf, nsem)

        drain(g, buf, sem)

    def pair(t, carry):
        step(2 * t, buf0, sem0, buf1, sem1)
        step(2 * t + 1, buf1, sem1, buf0, sem0)
        return carry

    lax.fori_loop(0, IPW // 2, pair, 0)


@jax.jit
def _lookup(inputs_t, tab_flat):
    mesh = plsc.VectorSubcoreMesh(core_axis_name="c", subcore_axis_name="s")
    run = pl.kernel(
        _sc_body,
        out_type=jax.ShapeDtypeStruct((F, D, B), jnp.float32),
        mesh=mesh,
        scratch_types=[
            pltpu.VMEM((2 * B,), jnp.int32),
            pltpu.VMEM((TAB,), jnp.float32),
            pltpu.VMEM((D, BBLK), jnp.float32),
            pltpu.VMEM((D, BBLK), jnp.float32),
            pltpu.SemaphoreType.DMA,
            pltpu.SemaphoreType.DMA,
        ],
        compiler_params=pltpu.CompilerParams(
            use_tc_tiling_on_sc=True, needs_layout_passes=False),
    )
    return run(inputs_t, tab_flat)


def kernel(inputs, tables):
    inputs_t = inputs.T
    tab_flat = tables.transpose(2, 0, 1).reshape(TAB)
    out_t = _lookup(inputs_t, tab_flat)
    return out_t.transpose(2, 0, 1)


# speedup vs baseline: 1.2464x; 1.2294x over previous
import jax
import jax.numpy as jnp
from jax import lax
from jax.experimental import pallas as pl
from jax.experimental.pallas import tpu as pltpu
from jax.experimental.pallas import tpu_sc as plsc

F = 26
V = 50
D = 32
B = 16384

NC = 2
NS = 16
NW = NC * NS
BBLK = 128
BPF = B // BBLK
ITEMS = F * BPF
IPW = ITEMS // NW
TAB = D * F * V
LANES = 16


def _sc_body(inputs_t_hbm, tab_hbm, out_hbm, idx_v, tab_v, buf0, buf1,
             sem0, sem1):
    wid = lax.axis_index("s") * NC + lax.axis_index("c")
    g0 = wid * IPW

    f_lo = g0 // BPF
    f_hi = (g0 + IPW - 1) // BPF
    pltpu.async_copy(tab_hbm, tab_v, sem0)
    pltpu.async_copy(inputs_t_hbm.at[f_lo], idx_v.at[pl.ds(0, B)], sem1)
    pltpu.async_copy(inputs_t_hbm.at[f_hi], idx_v.at[pl.ds(B, B)], sem1)
    pltpu.make_async_copy(tab_hbm, tab_v, sem0).wait()
    pltpu.make_async_copy(inputs_t_hbm.at[f_lo], idx_v.at[pl.ds(0, B)],
                          sem1).wait()
    pltpu.make_async_copy(inputs_t_hbm.at[f_hi], idx_v.at[pl.ds(B, B)],
                          sem1).wait()

    def compute(g, buf):
        f = g // BPF
        b0 = (g % BPF) * BBLK
        base_off = (f - f_lo) * B + b0
        fv = f * V
        @plsc.parallel_loop(0, BBLK // LANES, 1, unroll=2)
        def _grp(i):
            a16 = idx_v[pl.ds(base_off + LANES * i, LANES)] + fv
            for dd in range(0, D, 8):
                vals = [plsc.load_gather(tab_v, [a16 + (dd + k) * (F * V)])
                        for k in range(8)]
                for k in range(8):
                    buf[dd + k, pl.ds(LANES * i, LANES)] = vals[k]

    def fire(g, buf, sem):
        f = g // BPF
        b0 = (g % BPF) * BBLK
        pltpu.async_copy(buf, out_hbm.at[f, :, pl.ds(b0, BBLK)], sem)

    def drain(g, buf, sem):
        f = g // BPF
        b0 = (g % BPF) * BBLK
        pltpu.make_async_copy(buf, out_hbm.at[f, :, pl.ds(b0, BBLK)],
                              sem).wait()

    compute(g0, buf0)
    fire(g0, buf0, sem0)

    def step(t, buf, sem, nbuf, nsem):
        g = g0 + t

        @pl.when(t + 1 < IPW)
        def _():
            compute(g + 1, nbuf)
            fire(g + 1, nbuf, nsem)

        drain(g, buf, sem)

    def pair(t, carry):
        step(2 * t, buf0, sem0, buf1, sem1)
        step(2 * t + 1, buf1, sem1, buf0, sem0)
        return carry

    lax.fori_loop(0, IPW // 2, pair, 0)


@jax.jit
def _lookup(inputs_t, tab_flat):
    mesh = plsc.VectorSubcoreMesh(core_axis_name="c", subcore_axis_name="s")
    run = pl.kernel(
        _sc_body,
        out_type=jax.ShapeDtypeStruct((F, D, B), jnp.float32),
        mesh=mesh,
        scratch_types=[
            pltpu.VMEM((2 * B,), jnp.int32),
            pltpu.VMEM((TAB,), jnp.float32),
            pltpu.VMEM((D, BBLK), jnp.float32),
            pltpu.VMEM((D, BBLK), jnp.float32),
            pltpu.SemaphoreType.DMA,
            pltpu.SemaphoreType.DMA,
        ],
        compiler_params=pltpu.CompilerParams(
            use_tc_tiling_on_sc=True, needs_layout_passes=False),
    )
    return run(inputs_t, tab_flat)


def kernel(inputs, tables):
    inputs_t = inputs.T
    tab_flat = tables.transpose(2, 0, 1).reshape(TAB)
    out_t = _lookup(inputs_t, tab_flat)
    return out_t.transpose(2, 0, 1)


# speedup vs baseline: 1.2886x; 1.0339x over previous
import jax
import jax.numpy as jnp
from jax import lax
from jax.experimental import pallas as pl
from jax.experimental.pallas import tpu as pltpu
from jax.experimental.pallas import tpu_sc as plsc

F = 26
V = 50
D = 32
B = 16384

NC = 2
NS = 16
NW = NC * NS
BBLK = 128
BPF = B // BBLK
ITEMS = F * BPF
IPW = ITEMS // NW
TAB = D * F * V
LANES = 16


def _sc_body(inputs_t_hbm, tab_hbm, out_hbm, idx_v, tab_v, buf0, buf1,
             sem0, sem1):
    wid = lax.axis_index("s") * NC + lax.axis_index("c")
    g0 = wid * IPW

    f_lo = g0 // BPF
    f_hi = (g0 + IPW - 1) // BPF
    pltpu.async_copy(tab_hbm, tab_v, sem0)
    pltpu.async_copy(inputs_t_hbm.at[f_lo], idx_v.at[pl.ds(0, B)], sem1)
    pltpu.async_copy(inputs_t_hbm.at[f_hi], idx_v.at[pl.ds(B, B)], sem1)
    pltpu.make_async_copy(tab_hbm, tab_v, sem0).wait()
    pltpu.make_async_copy(inputs_t_hbm.at[f_lo], idx_v.at[pl.ds(0, B)],
                          sem1).wait()
    pltpu.make_async_copy(inputs_t_hbm.at[f_hi], idx_v.at[pl.ds(B, B)],
                          sem1).wait()

    def compute(g, buf):
        f = g // BPF
        b0 = (g % BPF) * BBLK
        base_off = (f - f_lo) * B + b0
        fv = f * V
        @plsc.parallel_loop(0, BBLK // LANES, 1, unroll=4)
        def _grp(i):
            a16 = idx_v[pl.ds(base_off + LANES * i, LANES)] + fv
            for dd in range(0, D, 8):
                vals = [plsc.load_gather(tab_v, [a16 + (dd + k) * (F * V)])
                        for k in range(8)]
                for k in range(8):
                    buf[dd + k, pl.ds(LANES * i, LANES)] = vals[k]

    def fire(g, buf, sem):
        f = g // BPF
        b0 = (g % BPF) * BBLK
        pltpu.async_copy(buf, out_hbm.at[f, :, pl.ds(b0, BBLK)], sem)

    def drain(g, buf, sem):
        f = g // BPF
        b0 = (g % BPF) * BBLK
        pltpu.make_async_copy(buf, out_hbm.at[f, :, pl.ds(b0, BBLK)],
                              sem).wait()

    compute(g0, buf0)
    fire(g0, buf0, sem0)

    def step(t, buf, sem, nbuf, nsem):
        g = g0 + t

        @pl.when(t + 1 < IPW)
        def _():
            compute(g + 1, nbuf)
            fire(g + 1, nbuf, nsem)

        drain(g, buf, sem)

    def pair(t, carry):
        step(2 * t, buf0, sem0, buf1, sem1)
        step(2 * t + 1, buf1, sem1, buf0, sem0)
        return carry

    lax.fori_loop(0, IPW // 2, pair, 0)


@jax.jit
def _lookup(inputs_t, tab_flat):
    mesh = plsc.VectorSubcoreMesh(core_axis_name="c", subcore_axis_name="s")
    run = pl.kernel(
        _sc_body,
        out_type=jax.ShapeDtypeStruct((F, D, B), jnp.float32),
        mesh=mesh,
        scratch_types=[
            pltpu.VMEM((2 * B,), jnp.int32),
            pltpu.VMEM((TAB,), jnp.float32),
            pltpu.VMEM((D, BBLK), jnp.float32),
            pltpu.VMEM((D, BBLK), jnp.float32),
            pltpu.SemaphoreType.DMA,
            pltpu.SemaphoreType.DMA,
        ],
        compiler_params=pltpu.CompilerParams(
            use_tc_tiling_on_sc=True, needs_layout_passes=False),
    )
    return run(inputs_t, tab_flat)


def kernel(inputs, tables):
    inputs_t = inputs.T
    tab_flat = tables.transpose(2, 0, 1).reshape(TAB)
    out_t = _lookup(inputs_t, tab_flat)
    return out_t.transpose(2, 0, 1)
